# Initial kernel scaffold; baseline (speedup 1.0000x reference)
import jax
import jax.numpy as jnp
from jax.experimental import pallas as pl


def kernel(x, edge_index, W1, b1, W2, b2):
    raise NotImplementedError("write your pallas kernel here")


# speedup vs baseline: 31.7542x; 31.7542x over previous
import functools

import jax
import jax.numpy as jnp
from jax import lax
from jax.experimental import pallas as pl
from jax.experimental.pallas import tpu as pltpu
from jax.experimental.pallas import tpu_sc as plsc

N = 10000
E = 320000
D = 128
NTILES = 32
CH = 128
NCH = 80
EPT = NCH * CH
EPAD = EPT * NTILES
NP = 10240
GRP = 8
NGRP = NCH // GRP
ZR = 160
NPC = 10240

_mesh = plsc.VectorSubcoreMesh(
    core_axis_name="c", subcore_axis_name="s", num_cores=2, num_subcores=16)

def _count_body(edges_hbm, out_hbm, dst_v, ones_v, zc_v, cnt, sem):
    c = lax.axis_index("c")
    s = lax.axis_index("s")
    wid = c * 16 + s

    pltpu.sync_copy(edges_hbm.at[1, pl.ds(wid * NCH, NCH), :], dst_v)

    zero16 = jnp.zeros((16,), jnp.float32)

    def _fill(i, _):
        zc_v[pl.ds(i * 16, 16)] = zero16
        return 0
    lax.fori_loop(0, 640 // 16, _fill, 0, unroll=8)
    for j in range(CH // 16):
        ones_v[pl.ds(j * 16, 16)] = zero16 + 1.0

    pltpu.sync_copy(zc_v, cnt.at[pl.ds(s * 640, 640)])
    plsc.subcore_barrier()

    def _body(g, _):
        pltpu.sync_copy(ones_v, cnt.at[dst_v.at[g]], add=True)
        return 0
    lax.fori_loop(0, NCH, _body, 0)

    plsc.subcore_barrier()
    pltpu.sync_copy(cnt.at[pl.ds(s * 640, 640)],
                    out_hbm.at[pl.ds(c * NPC + s * 640, 640)])


@functools.partial(
    pl.kernel,
    out_type=jax.ShapeDtypeStruct((2 * NPC,), jnp.float32),
    mesh=_mesh,
    scratch_types=[
        pltpu.VMEM((NCH, CH), jnp.int32),
        pltpu.VMEM((CH,), jnp.float32),
        pltpu.VMEM((640,), jnp.float32),
        pltpu.VMEM_SHARED((NPC,), jnp.float32),
        pltpu.SemaphoreType.DMA,
    ],
)
def _count_kernel(edges_hbm, out_hbm, dst_v, ones_v, zc_v, cnt, sem):
    _count_body(edges_hbm, out_hbm, dst_v, ones_v, zc_v, cnt, sem)


def _scatter_body(y_hbm, edges_hbm, out_hbm, dst_v, src_a, src_b, rows_a,
                  rows_b, acc, sem_ga, sem_gb, sem_ia, sem_ib):
    c = lax.axis_index("c")
    s = lax.axis_index("s")
    wid = c * 16 + s
    base = wid * NCH

    pltpu.sync_copy(edges_hbm.at[1, pl.ds(base, NCH), :], dst_v)

    zero16 = jnp.zeros((16,), jnp.float32)

    def _fill(i, _):
        for j in range(D // 16):
            rows_a[i, pl.ds(j * 16, 16)] = zero16
        return 0
    lax.fori_loop(0, CH, _fill, 0, unroll=4)
    for k in range(640 // CH):
        pltpu.sync_copy(rows_a, acc.at[pl.ds(s * 640 + k * CH, CH), :])
    plsc.subcore_barrier()

    rows = (rows_a, rows_b)
    gsems = (sem_ga, sem_gb)

    pltpu.sync_copy(edges_hbm.at[0, pl.ds(base, GRP), :], src_a)
    pltpu.async_copy(y_hbm.at[src_a.at[0]], rows_a, sem_ga)

    def _group(k, cur, nxt, sem_nxt):
        nxt_slab = edges_hbm.at[0, pl.ds(base + (k + 1) * GRP, GRP), :]

        @pl.when(k < NGRP - 1)
        def _():
            pltpu.async_copy(nxt_slab, nxt, sem_nxt)

        for j in range(GRP):
            g = k * GRP + j
            rcur, scur = rows[j % 2], gsems[j % 2]
            rnxt, snxt = rows[(j + 1) % 2], gsems[(j + 1) % 2]
            pltpu.make_async_copy(y_hbm.at[cur.at[j]], rcur, scur).wait()
            if j < GRP - 1:
                pltpu.async_copy(y_hbm.at[cur.at[j + 1]], rnxt, snxt)
            else:
                @pl.when(k < NGRP - 1)
                def _():
                    pltpu.make_async_copy(nxt_slab, nxt, sem_nxt).wait()
                    pltpu.async_copy(y_hbm.at[nxt.at[0]], rnxt, snxt)
            pltpu.sync_copy(rcur, acc.at[dst_v.at[g]], add=True)

    def _pair(t, _):
        _group(2 * t, src_a, src_b, sem_ib)
        _group(2 * t + 1, src_b, src_a, sem_ia)
        return 0
    lax.fori_loop(0, NGRP // 2, _pair, 0)

    plsc.subcore_barrier()
    pltpu.sync_copy(acc.at[pl.ds(s * 640, 640), :],
                    out_hbm.at[c, pl.ds(s * 640, 640), :])


@functools.partial(
    pl.kernel,
    out_type=jax.ShapeDtypeStruct((2, NP, D), jnp.float32),
    mesh=_mesh,
    scratch_types=[
        pltpu.VMEM((NCH, CH), jnp.int32),
        pltpu.VMEM((GRP, CH), jnp.int32),
        pltpu.VMEM((GRP, CH), jnp.int32),
        pltpu.VMEM((CH, D), jnp.float32),
        pltpu.VMEM((CH, D), jnp.float32),
        pltpu.VMEM_SHARED((NP, D), jnp.float32),
        pltpu.SemaphoreType.DMA,
        pltpu.SemaphoreType.DMA,
        pltpu.SemaphoreType.DMA,
        pltpu.SemaphoreType.DMA,
    ],
)
def _scatter_kernel(y_hbm, edges_hbm, out_hbm, dst_v, src_a, src_b, rows_a,
                    rows_b, acc, sem_ga, sem_gb, sem_ia, sem_ib):
    _scatter_body(y_hbm, edges_hbm, out_hbm, dst_v, src_a, src_b, rows_a,
                  rows_b, acc, sem_ga, sem_gb, sem_ia, sem_ib)


def _tc1_body(cnt_ref, x_ref, w_ref, dinv_ref, y_ref):
    cnt = cnt_ref[...]
    deg = cnt[0, :N] + cnt[1, :N] + 1.0
    dinv = lax.rsqrt(deg)[:, None]
    dinv_ref[...] = dinv
    xw = jnp.dot(x_ref[...], w_ref[...], preferred_element_type=jnp.float32)
    y_ref[...] = dinv * xw


def _tc1(cnt, x, w1):
    return pl.pallas_call(
        _tc1_body,
        out_shape=(jax.ShapeDtypeStruct((N, 1), jnp.float32),
                   jax.ShapeDtypeStruct((N, D), jnp.float32)),
    )(cnt, x, w1)


def _tc2_body(s_ref, y_ref, dinv_ref, b_ref, w_ref, y2_ref):
    dinv = dinv_ref[...]
    h = dinv * (s_ref[0, :N, :] + s_ref[1, :N, :] + y_ref[...]) + b_ref[...]
    h = jnp.maximum(h, 0.0)
    y2_ref[...] = dinv * jnp.dot(h, w_ref[...],
                                 preferred_element_type=jnp.float32)


def _tc2(s1, y1, dinv, b1, w2):
    return pl.pallas_call(
        _tc2_body,
        out_shape=jax.ShapeDtypeStruct((N, D), jnp.float32),
    )(s1, y1, dinv, b1, w2)


def _tc3_body(s_ref, y_ref, dinv_ref, b_ref, out_ref):
    dinv = dinv_ref[...]
    out_ref[...] = (dinv * (s_ref[0, :N, :] + s_ref[1, :N, :] + y_ref[...])
                    + b_ref[...])


def _tc3(s2, y2, dinv, b2):
    return pl.pallas_call(
        _tc3_body,
        out_shape=jax.ShapeDtypeStruct((N, D), jnp.float32),
    )(s2, y2, dinv, b2)


def kernel(x, edge_index, W1, b1, W2, b2):
    k = jnp.arange(EPAD - E, dtype=jnp.int32)
    pad = jnp.stack([(k * 131) % N, N + (k % 32)])
    edges = jnp.concatenate([edge_index, pad], axis=1)
    edges = edges.reshape(2, NTILES * NCH, CH)

    cnt = _count_kernel(edges).reshape(2, NPC)
    dinv, y1 = _tc1(cnt, x, W1)
    s1 = _scatter_kernel(y1, edges)
    y2 = _tc2(s1, y1, dinv, b1, W2)
    s2 = _scatter_kernel(y2, edges)
    out = _tc3(s2, y2, dinv, b2)
    return out
